# baseline (speedup 1.0000x reference)
import functools

import jax
import jax.numpy as jnp
from jax import lax
from jax.experimental import pallas as pl
from jax.experimental.pallas import tpu as pltpu
from jax.experimental.pallas import tpu_sc as plsc

B = 1024
T = 2048
D = 64
DP = 128
L = 16
NC = 2
NS = 16
NW = NC * NS
NTS = 16
TS = T // NTS
NBH = NW // NTS
BH = B // NBH
BLK = 8
NBLK = BH // BLK
NBUF = 4
VOCAB = 1000000
RB = 8000


def _pad_block(x_ref, o_ref):
    o_ref[...] = jnp.concatenate([x_ref[...], x_ref[...]], axis=1)


_tok_pad = pl.pallas_call(
    _pad_block,
    grid=(VOCAB // RB,),
    in_specs=[pl.BlockSpec((RB, D), lambda i: (i, 0))],
    out_specs=pl.BlockSpec((RB, DP), lambda i: (i, 0)),
    out_shape=jax.ShapeDtypeStruct((VOCAB, DP), jnp.float32),
)

def _emb_body(idx_hbm, pad_hbm, out_hbm, idx_v,
              rbufs, gsems, wsems, isem):
    wid = lax.axis_index("s") * NC + lax.axis_index("c")
    t0 = (wid % NTS) * TS
    bbase = (wid // NTS) * BH

    def gather_desc(blksel, j, s):
        return pltpu.make_async_copy(
            pad_hbm.at[idx_v.at[blksel, j]], rbufs[s], gsems[s])

    def write_desc(p, s):
        return pltpu.make_async_copy(
            rbufs[s],
            out_hbm.at[pl.ds((bbase + p) * T + t0, TS)],
            wsems[s])

    def idx_desc(blk, sel):
        return pltpu.make_async_copy(
            idx_hbm.at[pl.ds(bbase + blk * BLK, BLK), pl.ds(t0, TS)],
            idx_v.at[sel], isem)

    idx_desc(0, 0).start()
    idx_desc(0, 0).wait()
    gather_desc(0, 0, 0).start()
    gather_desc(0, 1, 1).start()

    def block_body(i, carry):
        isel = i % 2
        nsel = (i + 1) % 2
        not_last = i < NBLK - 1

        for ph in range(BLK):
            p = i * BLK + ph
            s = ph % NBUF
            s2 = (ph + 2) % NBUF

            if ph == 0:
                @pl.when(not_last)
                def _():
                    idx_desc(i + 1, nsel).start()

            if ph < 2:
                @pl.when(i > 0)
                def _():
                    write_desc(p - 2, s2).wait()
            else:
                write_desc(p - 2, s2).wait()

            if ph == BLK - 2:
                @pl.when(not_last)
                def _():
                    idx_desc(i + 1, nsel).wait()

            if ph < BLK - 2:
                gather_desc(isel, ph + 2, s2).start()
            else:
                @pl.when(not_last)
                def _():
                    gather_desc(nsel, ph + 2 - BLK, s2).start()

            gather_desc(isel, ph, s).wait()
            write_desc(p, s).start()

        return carry

    lax.fori_loop(0, NBLK, block_body, 0)

    last = NBLK * BLK
    write_desc(last - 2, 2).wait()
    write_desc(last - 1, 3).wait()


@functools.partial(
    pl.kernel,
    out_type=jax.ShapeDtypeStruct((B * T, DP), jnp.float32),
    mesh=plsc.VectorSubcoreMesh(core_axis_name="c", subcore_axis_name="s"),
    compiler_params=pltpu.CompilerParams(use_tc_tiling_on_sc=False),
    scratch_types=[
        pltpu.VMEM((2, BLK, TS), jnp.int32),
        pltpu.VMEM((TS, DP), jnp.float32),
        pltpu.VMEM((TS, DP), jnp.float32),
        pltpu.VMEM((TS, DP), jnp.float32),
        pltpu.VMEM((TS, DP), jnp.float32),
        pltpu.SemaphoreType.DMA,
        pltpu.SemaphoreType.DMA,
        pltpu.SemaphoreType.DMA,
        pltpu.SemaphoreType.DMA,
        pltpu.SemaphoreType.DMA,
        pltpu.SemaphoreType.DMA,
        pltpu.SemaphoreType.DMA,
        pltpu.SemaphoreType.DMA,
        pltpu.SemaphoreType.DMA,
    ],
)
def _emb_call(idx_hbm, pad_hbm, out_hbm, idx_v,
              rb0, rb1, rb2, rb3,
              g0, g1, g2, g3, w0, w1, w2, w3, isem):
    _emb_body(idx_hbm, pad_hbm, out_hbm, idx_v,
              [rb0, rb1, rb2, rb3],
              [g0, g1, g2, g3], [w0, w1, w2, w3], isem)


BT2 = 512


def _slice_add_block(x_ref, p_ref, o_ref):
    o_ref[...] = (x_ref[:, :D] + p_ref[...])[None]


_slice_add = pl.pallas_call(
    _slice_add_block,
    grid=(T // BT2, B),
    in_specs=[
        pl.BlockSpec((BT2, DP), lambda ti, bj: (bj * (T // BT2) + ti, 0)),
        pl.BlockSpec((BT2, D), lambda ti, bj: (ti, 0)),
    ],
    out_specs=pl.BlockSpec((1, BT2, D), lambda ti, bj: (bj, ti, 0)),
    out_shape=jax.ShapeDtypeStruct((B, T, D), jnp.float32),
)


def kernel(idx, tok_emb, pos_emb):
    tok_pad = _tok_pad(tok_emb)
    gathered = _emb_call(idx, tok_pad)
    return _slice_add(gathered, pos_emb)


# speedup vs baseline: 2.3415x; 2.3415x over previous
import functools

import jax
import jax.numpy as jnp
from jax import lax
from jax.experimental import pallas as pl
from jax.experimental.pallas import tpu as pltpu
from jax.experimental.pallas import tpu_sc as plsc

B = 1024
T = 2048
D = 64
DP = 128
L = 16
NC = 2
NS = 16
NW = NC * NS
NTS = 16
TS = T // NTS
NBH = NW // NTS
BH = B // NBH
BLK = 8
NBLK = BH // BLK
NBUF = 4
VOCAB = 1000000
RB = 8000


def _pad_block(x_ref, o_ref):
    o_ref[...] = jnp.concatenate([x_ref[...], x_ref[...]], axis=1)


_tok_pad = pl.pallas_call(
    _pad_block,
    grid=(VOCAB // RB,),
    in_specs=[pl.BlockSpec((RB, D), lambda i: (i, 0))],
    out_specs=pl.BlockSpec((RB, DP), lambda i: (i, 0)),
    out_shape=jax.ShapeDtypeStruct((VOCAB, DP), jnp.float32),
)

def _emb_body(idx_hbm, pad_hbm, pos_hbm, out_hbm, pos_v, idx_v,
              rbufs, gsems, wsems, isem):
    wid = lax.axis_index("s") * NC + lax.axis_index("c")
    t0 = (wid % NTS) * TS
    bbase = (wid // NTS) * BH

    pltpu.sync_copy(pos_hbm.at[pl.ds(t0, TS), pl.ds(0, D)], pos_v)

    def gather_desc(blksel, j, s):
        return pltpu.make_async_copy(
            pad_hbm.at[idx_v.at[blksel, j]], rbufs[s], gsems[s])

    def write_desc(p, s):
        return pltpu.make_async_copy(
            rbufs[s],
            out_hbm.at[pl.ds((bbase + p) * T + t0, TS)],
            wsems[s])

    def idx_desc(blk, sel):
        return pltpu.make_async_copy(
            idx_hbm.at[pl.ds(bbase + blk * BLK, BLK), pl.ds(t0, TS)],
            idx_v.at[sel], isem)

    def add_pos(s):
        def row_body(r, rc):
            for k in range(D // L):
                sl = pl.ds(k * L, L)
                plsc.addupdate(rbufs[s].at[r, sl], pos_v[r, sl])
            return rc
        lax.fori_loop(0, TS, row_body, 0)

    idx_desc(0, 0).start()
    idx_desc(0, 0).wait()
    gather_desc(0, 0, 0).start()
    gather_desc(0, 1, 1).start()

    def block_body(i, carry):
        isel = i % 2
        nsel = (i + 1) % 2
        not_last = i < NBLK - 1

        for ph in range(BLK):
            p = i * BLK + ph
            s = ph % NBUF
            s2 = (ph + 2) % NBUF

            if ph == 0:
                @pl.when(not_last)
                def _():
                    idx_desc(i + 1, nsel).start()

            if ph < 2:
                @pl.when(i > 0)
                def _():
                    write_desc(p - 2, s2).wait()
            else:
                write_desc(p - 2, s2).wait()

            if ph == BLK - 2:
                @pl.when(not_last)
                def _():
                    idx_desc(i + 1, nsel).wait()

            if ph < BLK - 2:
                gather_desc(isel, ph + 2, s2).start()
            else:
                @pl.when(not_last)
                def _():
                    gather_desc(nsel, ph + 2 - BLK, s2).start()

            gather_desc(isel, ph, s).wait()
            add_pos(s)
            write_desc(p, s).start()

        return carry

    lax.fori_loop(0, NBLK, block_body, 0)

    last = NBLK * BLK
    write_desc(last - 2, 2).wait()
    write_desc(last - 1, 3).wait()


@functools.partial(
    pl.kernel,
    out_type=jax.ShapeDtypeStruct((B * T, DP), jnp.float32),
    mesh=plsc.VectorSubcoreMesh(core_axis_name="c", subcore_axis_name="s"),
    compiler_params=pltpu.CompilerParams(use_tc_tiling_on_sc=False),
    scratch_types=[
        pltpu.VMEM((TS, D), jnp.float32),
        pltpu.VMEM((2, BLK, TS), jnp.int32),
        pltpu.VMEM((TS, DP), jnp.float32),
        pltpu.VMEM((TS, DP), jnp.float32),
        pltpu.VMEM((TS, DP), jnp.float32),
        pltpu.VMEM((TS, DP), jnp.float32),
        pltpu.SemaphoreType.DMA,
        pltpu.SemaphoreType.DMA,
        pltpu.SemaphoreType.DMA,
        pltpu.SemaphoreType.DMA,
        pltpu.SemaphoreType.DMA,
        pltpu.SemaphoreType.DMA,
        pltpu.SemaphoreType.DMA,
        pltpu.SemaphoreType.DMA,
        pltpu.SemaphoreType.DMA,
    ],
)
def _emb_call(idx_hbm, pad_hbm, pos_hbm, out_hbm, pos_v, idx_v,
              rb0, rb1, rb2, rb3,
              g0, g1, g2, g3, w0, w1, w2, w3, isem):
    _emb_body(idx_hbm, pad_hbm, pos_hbm, out_hbm, pos_v, idx_v,
              [rb0, rb1, rb2, rb3],
              [g0, g1, g2, g3], [w0, w1, w2, w3], isem)


_pos_pad = pl.pallas_call(
    _pad_block,
    grid=(1,),
    in_specs=[pl.BlockSpec((T, D), lambda i: (0, 0))],
    out_specs=pl.BlockSpec((T, DP), lambda i: (0, 0)),
    out_shape=jax.ShapeDtypeStruct((T, DP), jnp.float32),
)


def kernel(idx, tok_emb, pos_emb):
    tok_pad = _tok_pad(tok_emb)
    pos_pad = _pos_pad(pos_emb)
    out128 = _emb_call(idx, tok_pad, pos_pad)
    return out128.reshape(B, T, DP)[:, :, :D]


# speedup vs baseline: 2.3635x; 1.0094x over previous
import functools

import jax
import jax.numpy as jnp
from jax import lax
from jax.experimental import pallas as pl
from jax.experimental.pallas import tpu as pltpu
from jax.experimental.pallas import tpu_sc as plsc

B = 1024
T = 2048
D = 64
DP = 128
L = 16
NC = 2
NS = 16
NW = NC * NS
NTS = 16
TS = T // NTS
NBH = NW // NTS
BH = B // NBH
BLK = 8
NBLK = BH // BLK
NBUF = 4
VOCAB = 1000000
RB = 8000


def _pad_block(x_ref, o_ref):
    o_ref[...] = jnp.concatenate([x_ref[...], x_ref[...]], axis=1)


_tok_pad = pl.pallas_call(
    _pad_block,
    grid=(VOCAB // RB,),
    in_specs=[pl.BlockSpec((RB, D), lambda i: (i, 0))],
    out_specs=pl.BlockSpec((RB, DP), lambda i: (i, 0)),
    out_shape=jax.ShapeDtypeStruct((VOCAB, DP), jnp.float32),
)

def _emb_body(idx_hbm, pad_hbm, pos_hbm, out_hbm, pos_v, idx_v,
              rbufs, gsems, wsems, isem):
    wid = lax.axis_index("s") * NC + lax.axis_index("c")
    t0 = (wid % NTS) * TS
    bbase = (wid // NTS) * BH

    pltpu.sync_copy(pos_hbm.at[pl.ds(t0, TS), pl.ds(0, D)], pos_v)

    def gather_desc(blksel, j, s):
        return pltpu.make_async_copy(
            pad_hbm.at[idx_v.at[blksel, j]], rbufs[s], gsems[s])

    def write_desc(p, s):
        return pltpu.make_async_copy(
            rbufs[s],
            out_hbm.at[pl.ds((bbase + p) * T + t0, TS)],
            wsems[s])

    def idx_desc(blk, sel):
        return pltpu.make_async_copy(
            idx_hbm.at[pl.ds(bbase + blk * BLK, BLK), pl.ds(t0, TS)],
            idx_v.at[sel], isem)

    def add_pos(s):
        def row_body(g, rc):
            r0 = g * 4
            for u in range(4):
                for k in range(D // L):
                    sl = pl.ds(k * L, L)
                    plsc.addupdate(rbufs[s].at[r0 + u, sl], pos_v[r0 + u, sl])
            return rc
        lax.fori_loop(0, TS // 4, row_body, 0)

    idx_desc(0, 0).start()
    idx_desc(0, 0).wait()
    gather_desc(0, 0, 0).start()
    gather_desc(0, 1, 1).start()

    def block_body(i, carry):
        isel = i % 2
        nsel = (i + 1) % 2
        not_last = i < NBLK - 1

        for ph in range(BLK):
            p = i * BLK + ph
            s = ph % NBUF
            s2 = (ph + 2) % NBUF

            if ph == 0:
                @pl.when(not_last)
                def _():
                    idx_desc(i + 1, nsel).start()

            if ph < 2:
                @pl.when(i > 0)
                def _():
                    write_desc(p - 2, s2).wait()
            else:
                write_desc(p - 2, s2).wait()

            if ph == BLK - 2:
                @pl.when(not_last)
                def _():
                    idx_desc(i + 1, nsel).wait()

            if ph < BLK - 2:
                gather_desc(isel, ph + 2, s2).start()
            else:
                @pl.when(not_last)
                def _():
                    gather_desc(nsel, ph + 2 - BLK, s2).start()

            gather_desc(isel, ph, s).wait()
            add_pos(s)
            write_desc(p, s).start()

        return carry

    lax.fori_loop(0, NBLK, block_body, 0)

    last = NBLK * BLK
    write_desc(last - 2, 2).wait()
    write_desc(last - 1, 3).wait()


@functools.partial(
    pl.kernel,
    out_type=jax.ShapeDtypeStruct((B * T, DP), jnp.float32),
    mesh=plsc.VectorSubcoreMesh(core_axis_name="c", subcore_axis_name="s"),
    compiler_params=pltpu.CompilerParams(use_tc_tiling_on_sc=False),
    scratch_types=[
        pltpu.VMEM((TS, D), jnp.float32),
        pltpu.VMEM((2, BLK, TS), jnp.int32),
        pltpu.VMEM((TS, DP), jnp.float32),
        pltpu.VMEM((TS, DP), jnp.float32),
        pltpu.VMEM((TS, DP), jnp.float32),
        pltpu.VMEM((TS, DP), jnp.float32),
        pltpu.SemaphoreType.DMA,
        pltpu.SemaphoreType.DMA,
        pltpu.SemaphoreType.DMA,
        pltpu.SemaphoreType.DMA,
        pltpu.SemaphoreType.DMA,
        pltpu.SemaphoreType.DMA,
        pltpu.SemaphoreType.DMA,
        pltpu.SemaphoreType.DMA,
        pltpu.SemaphoreType.DMA,
    ],
)
def _emb_call(idx_hbm, pad_hbm, pos_hbm, out_hbm, pos_v, idx_v,
              rb0, rb1, rb2, rb3,
              g0, g1, g2, g3, w0, w1, w2, w3, isem):
    _emb_body(idx_hbm, pad_hbm, pos_hbm, out_hbm, pos_v, idx_v,
              [rb0, rb1, rb2, rb3],
              [g0, g1, g2, g3], [w0, w1, w2, w3], isem)


_pos_pad = pl.pallas_call(
    _pad_block,
    grid=(1,),
    in_specs=[pl.BlockSpec((T, D), lambda i: (0, 0))],
    out_specs=pl.BlockSpec((T, DP), lambda i: (0, 0)),
    out_shape=jax.ShapeDtypeStruct((T, DP), jnp.float32),
)


def kernel(idx, tok_emb, pos_emb):
    tok_pad = _tok_pad(tok_emb)
    pos_pad = _pos_pad(pos_emb)
    out128 = _emb_call(idx, tok_pad, pos_pad)
    return out128.reshape(B, T, DP)[:, :, :D]
